# baseline (speedup 1.0000x reference)
import jax
import jax.numpy as jnp
from jax import lax
from jax.experimental import pallas as pl
from jax.experimental.pallas import tpu as pltpu
from jax.experimental.pallas import tpu_sc as plsc

BATCH = 4
SEQ_LEN = 4096
D_MODEL = 1024
NUM_CORES = 2
NUM_SUBCORES = 16
NUM_WORKERS = NUM_CORES * NUM_SUBCORES
S_PER_WORKER = SEQ_LEN // NUM_WORKERS
CHUNK = 16
NUM_CHUNKS = S_PER_WORKER // CHUNK
NUM_UNITS = NUM_CHUNKS * BATCH
NBUF = 5
LANES = 16
VECS_PER_ROW = D_MODEL // LANES


def _body(idx_hbm, table_hbm, pos_hbm, out_hbm,
          idx_v, pos0, pos1, rows0, rows1, rows2, rows3, rows4,
          sg0, sg1, sg2, sg3, sg4, ss0, ss1, ss2, ss3, ss4, sp0, sp1):
    wid = lax.axis_index("c") * NUM_SUBCORES + lax.axis_index("s")
    s_base = wid * S_PER_WORKER

    pos_b = [pos0, pos1]
    rows_b = [rows0, rows1, rows2, rows3, rows4]
    sg = [sg0, sg1, sg2, sg3, sg4]
    ss = [ss0, ss1, ss2, ss3, ss4]
    sp = [sp0, sp1]

    def idx_slice(u):
        c, b = divmod(u, BATCH)
        return idx_v.at[b, pl.ds(c * CHUNK, CHUNK)]

    pos_cp = [None] * NUM_CHUNKS
    gath = [None] * NUM_UNITS
    stor = [None] * NUM_UNITS

    pos_cp[0] = pltpu.async_copy(pos_hbm.at[pl.ds(s_base, CHUNK)], pos_b[0], sp[0])
    pltpu.sync_copy(idx_hbm.at[:, pl.ds(s_base, S_PER_WORKER)], idx_v)
    gath[0] = pltpu.async_copy(table_hbm.at[idx_slice(0)], rows_b[0], sg[0])
    gath[1] = pltpu.async_copy(table_hbm.at[idx_slice(1)], rows_b[1], sg[1])
    gath[2] = pltpu.async_copy(table_hbm.at[idx_slice(2)], rows_b[2], sg[2])

    for u in range(NUM_UNITS):
        c, b = divmod(u, BATCH)
        pr = u % NBUF
        pc = c & 1
        if b == 0 and c + 1 < NUM_CHUNKS:
            pos_cp[c + 1] = pltpu.async_copy(
                pos_hbm.at[pl.ds(s_base + (c + 1) * CHUNK, CHUNK)],
                pos_b[pc ^ 1], sp[pc ^ 1])
        v = u + 3
        if v < NUM_UNITS:
            if v - NBUF >= 0:
                for d in stor[v - NBUF]:
                    d.wait()
            gath[v] = pltpu.async_copy(
                table_hbm.at[idx_slice(v)],
                rows_b[v % NBUF], sg[v % NBUF])
        gath[u].wait()
        if b == 0:
            pos_cp[c].wait()

        st0 = s_base + c * CHUNK

        def row_add(i, carry):
            for j in range(VECS_PER_ROW):
                x = pos_b[pc][i, pl.ds(j * LANES, LANES)]
                plsc.addupdate(rows_b[pr].at[i, pl.ds(j * LANES, LANES)], x)
            return carry

        lax.fori_loop(0, CHUNK, row_add, 0)
        stor[u] = [pltpu.async_copy(
            rows_b[pr], out_hbm.at[b, pl.ds(st0, CHUNK)], ss[pr])]

    for u in range(NUM_UNITS - NBUF + 1, NUM_UNITS):
        if u >= 0:
            for d in stor[u]:
                d.wait()


def kernel(inputs, tok_embeddings, learned_pos_enc):
    if inputs.dtype != jnp.int32:
        inputs = inputs.astype(jnp.int32)
    mesh = plsc.VectorSubcoreMesh(
        core_axis_name="c",
        subcore_axis_name="s",
        num_cores=NUM_CORES,
        num_subcores=NUM_SUBCORES,
    )
    k = pl.kernel(
        _body,
        out_type=jax.ShapeDtypeStruct((BATCH, SEQ_LEN, D_MODEL), jnp.float32),
        mesh=mesh,
        scratch_types=[
            pltpu.VMEM((BATCH, S_PER_WORKER), jnp.int32),
            pltpu.VMEM((CHUNK, D_MODEL), jnp.float32),
            pltpu.VMEM((CHUNK, D_MODEL), jnp.float32),
            pltpu.VMEM((CHUNK, D_MODEL), jnp.float32),
            pltpu.VMEM((CHUNK, D_MODEL), jnp.float32),
            pltpu.VMEM((CHUNK, D_MODEL), jnp.float32),
            pltpu.VMEM((CHUNK, D_MODEL), jnp.float32),
            pltpu.VMEM((CHUNK, D_MODEL), jnp.float32),
            pltpu.SemaphoreType.DMA,
            pltpu.SemaphoreType.DMA,
            pltpu.SemaphoreType.DMA,
            pltpu.SemaphoreType.DMA,
            pltpu.SemaphoreType.DMA,
            pltpu.SemaphoreType.DMA,
            pltpu.SemaphoreType.DMA,
            pltpu.SemaphoreType.DMA,
            pltpu.SemaphoreType.DMA,
            pltpu.SemaphoreType.DMA,
            pltpu.SemaphoreType.DMA,
            pltpu.SemaphoreType.DMA,
        ],
    )
    return k(inputs, tok_embeddings, learned_pos_enc)


# speedup vs baseline: 1.0030x; 1.0030x over previous
import jax
import jax.numpy as jnp
from jax import lax
from jax.experimental import pallas as pl
from jax.experimental.pallas import tpu as pltpu
from jax.experimental.pallas import tpu_sc as plsc

BATCH = 4
SEQ_LEN = 4096
D_MODEL = 1024
NUM_CORES = 2
NUM_SUBCORES = 16
NUM_WORKERS = NUM_CORES * NUM_SUBCORES
S_PER_WORKER = SEQ_LEN // NUM_WORKERS
CHUNK = 16
NUM_CHUNKS = S_PER_WORKER // CHUNK
NUM_UNITS = NUM_CHUNKS * BATCH
NBUF = 5
LANES = 16
VECS_PER_ROW = D_MODEL // LANES


def _body(idx_hbm, table_hbm, pos_hbm, out_hbm,
          idx_v, pos0, pos1, rows0, rows1, rows2, rows3, rows4,
          sg0, sg1, sg2, sg3, sg4, ss0, ss1, ss2, ss3, ss4, sp0, sp1):
    wid = lax.axis_index("s") * NUM_CORES + lax.axis_index("c")
    s_base = wid * S_PER_WORKER

    pos_b = [pos0, pos1]
    rows_b = [rows0, rows1, rows2, rows3, rows4]
    sg = [sg0, sg1, sg2, sg3, sg4]
    ss = [ss0, ss1, ss2, ss3, ss4]
    sp = [sp0, sp1]

    def idx_slice(u):
        c, b = divmod(u, BATCH)
        return idx_v.at[b, pl.ds(c * CHUNK, CHUNK)]

    pos_cp = [None] * NUM_CHUNKS
    gath = [None] * NUM_UNITS
    stor = [None] * NUM_UNITS

    pos_cp[0] = pltpu.async_copy(pos_hbm.at[pl.ds(s_base, CHUNK)], pos_b[0], sp[0])
    pltpu.sync_copy(idx_hbm.at[:, pl.ds(s_base, S_PER_WORKER)], idx_v)
    gath[0] = pltpu.async_copy(table_hbm.at[idx_slice(0)], rows_b[0], sg[0])
    gath[1] = pltpu.async_copy(table_hbm.at[idx_slice(1)], rows_b[1], sg[1])
    gath[2] = pltpu.async_copy(table_hbm.at[idx_slice(2)], rows_b[2], sg[2])

    for u in range(NUM_UNITS):
        c, b = divmod(u, BATCH)
        pr = u % NBUF
        pc = c & 1
        if b == 0 and c + 1 < NUM_CHUNKS:
            pos_cp[c + 1] = pltpu.async_copy(
                pos_hbm.at[pl.ds(s_base + (c + 1) * CHUNK, CHUNK)],
                pos_b[pc ^ 1], sp[pc ^ 1])
        v = u + 3
        if v < NUM_UNITS:
            if v - NBUF >= 0:
                for d in stor[v - NBUF]:
                    d.wait()
            gath[v] = pltpu.async_copy(
                table_hbm.at[idx_slice(v)],
                rows_b[v % NBUF], sg[v % NBUF])
        gath[u].wait()
        if b == 0:
            pos_cp[c].wait()

        st0 = s_base + c * CHUNK

        def row_add(i, carry):
            for j in range(VECS_PER_ROW):
                x = pos_b[pc][i, pl.ds(j * LANES, LANES)]
                plsc.addupdate(rows_b[pr].at[i, pl.ds(j * LANES, LANES)], x)
            return carry

        lax.fori_loop(0, CHUNK, row_add, 0)
        stor[u] = [pltpu.async_copy(
            rows_b[pr], out_hbm.at[b, pl.ds(st0, CHUNK)], ss[pr])]

    for u in range(NUM_UNITS - NBUF + 1, NUM_UNITS):
        if u >= 0:
            for d in stor[u]:
                d.wait()


def kernel(inputs, tok_embeddings, learned_pos_enc):
    if inputs.dtype != jnp.int32:
        inputs = inputs.astype(jnp.int32)
    mesh = plsc.VectorSubcoreMesh(
        core_axis_name="c",
        subcore_axis_name="s",
        num_cores=NUM_CORES,
        num_subcores=NUM_SUBCORES,
    )
    k = pl.kernel(
        _body,
        out_type=jax.ShapeDtypeStruct((BATCH, SEQ_LEN, D_MODEL), jnp.float32),
        mesh=mesh,
        scratch_types=[
            pltpu.VMEM((BATCH, S_PER_WORKER), jnp.int32),
            pltpu.VMEM((CHUNK, D_MODEL), jnp.float32),
            pltpu.VMEM((CHUNK, D_MODEL), jnp.float32),
            pltpu.VMEM((CHUNK, D_MODEL), jnp.float32),
            pltpu.VMEM((CHUNK, D_MODEL), jnp.float32),
            pltpu.VMEM((CHUNK, D_MODEL), jnp.float32),
            pltpu.VMEM((CHUNK, D_MODEL), jnp.float32),
            pltpu.VMEM((CHUNK, D_MODEL), jnp.float32),
            pltpu.SemaphoreType.DMA,
            pltpu.SemaphoreType.DMA,
            pltpu.SemaphoreType.DMA,
            pltpu.SemaphoreType.DMA,
            pltpu.SemaphoreType.DMA,
            pltpu.SemaphoreType.DMA,
            pltpu.SemaphoreType.DMA,
            pltpu.SemaphoreType.DMA,
            pltpu.SemaphoreType.DMA,
            pltpu.SemaphoreType.DMA,
            pltpu.SemaphoreType.DMA,
            pltpu.SemaphoreType.DMA,
        ],
    )
    return k(inputs, tok_embeddings, learned_pos_enc)
